# baseline (speedup 1.0000x reference)
import jax
import jax.numpy as jnp
from jax.experimental import pallas as pl

TOP_K = 256


def _noop_body(x_ref, o_ref):
    o_ref[...] = x_ref[...]


def kernel(x, filter_real, filter_imag):
    seq_len = x.shape[1]
    X_f = jnp.fft.rfft(x, axis=1)
    magnitudes = jnp.abs(X_f)
    k = min(TOP_K, magnitudes.shape[1])
    m_t = jnp.transpose(magnitudes, (0, 2, 1))
    _, idx_t = jax.lax.top_k(m_t, k)
    topk_indices = jnp.transpose(idx_t, (0, 2, 1))
    selected = jnp.take_along_axis(X_f, topk_indices, axis=1)
    cfilt = (filter_real + 1j * filter_imag).astype(jnp.complex64)
    filtered = selected * cfilt[None, :, :]
    B, F_, D = X_f.shape
    b_idx = jnp.arange(B)[:, None, None]
    d_idx = jnp.arange(D)[None, None, :]
    X_filtered = jnp.zeros_like(X_f).at[b_idx, topk_indices, d_idx].set(filtered)
    x_out = jnp.fft.irfft(X_filtered, n=seq_len, axis=1)
    x_out = pl.pallas_call(
        _noop_body,
        grid=(B, seq_len // 512),
        in_specs=[pl.BlockSpec((1, 512, D), lambda i, j: (i, j, 0))],
        out_specs=pl.BlockSpec((1, 512, D), lambda i, j: (i, j, 0)),
        out_shape=jax.ShapeDtypeStruct(x_out.shape, x_out.dtype),
    )(x_out)
    return x_out


# speedup vs baseline: 4.7013x; 4.7013x over previous
import jax
import jax.numpy as jnp
import numpy as np
from jax import lax
from jax.experimental import pallas as pl
from jax.experimental.pallas import tpu as pltpu

_N = 8192
_D = 768
_F = _N // 2 + 1
_FP = 4352
_K = 256

_HI = jax.lax.Precision.HIGHEST


def _np_bases():
    n = np.arange(_N, dtype=np.float64)
    f = np.arange(_FP, dtype=np.float64)
    ang = (2.0 * np.pi / _N) * np.outer(f, n)
    ct = np.cos(ang)
    st = -np.sin(ang)
    wr = np.full(_FP, 2.0)
    wr[0] = 1.0
    wr[_F - 1] = 1.0
    wr[_F:] = 0.0
    wi = np.full(_FP, 2.0)
    wi[0] = 0.0
    wi[_F - 1 :] = 0.0
    br = (wr[None, :] / _N) * np.cos(ang.T)
    bi = -(wi[None, :] / _N) * np.sin(ang.T)
    return (ct.astype(np.float32), st.astype(np.float32),
            br.astype(np.float32), bi.astype(np.float32))


_CT, _ST, _BR, _BI = _np_bases()

_DBLK = 128
_FBLK = 256
_NBLK = 512


def _fwd_body(ct_ref, st_ref, x_ref, xr_ref, xi_ref):
    x = x_ref[0]
    xr_ref[0] = jnp.dot(ct_ref[...], x, preferred_element_type=jnp.float32,
                        precision=_HI)
    xi_ref[0] = jnp.dot(st_ref[...], x, preferred_element_type=jnp.float32,
                        precision=_HI)


def _fwd_dft(x, ct, st):
    B = x.shape[0]
    return pl.pallas_call(
        _fwd_body,
        grid=(B, _FP // _FBLK, _D // _DBLK),
        in_specs=[
            pl.BlockSpec((_FBLK, _N), lambda b, f, d: (f, 0)),
            pl.BlockSpec((_FBLK, _N), lambda b, f, d: (f, 0)),
            pl.BlockSpec((1, _N, _DBLK), lambda b, f, d: (b, 0, d)),
        ],
        out_specs=[
            pl.BlockSpec((1, _FBLK, _DBLK), lambda b, f, d: (b, f, d)),
            pl.BlockSpec((1, _FBLK, _DBLK), lambda b, f, d: (b, f, d)),
        ],
        out_shape=[
            jax.ShapeDtypeStruct((B, _FP, _D), jnp.float32),
            jax.ShapeDtypeStruct((B, _FP, _D), jnp.float32),
        ],
    )(ct, st, x)


def _select_body(xr_ref, xi_ref, fr_ref, fi_ref, sr_ref, si_ref, mag_ref):
    xr = xr_ref[0]
    xi = xi_ref[0]
    fi_iota = lax.broadcasted_iota(jnp.int32, (_FP, _DBLK), 0)
    mag = jnp.sqrt(xr * xr + xi * xi)
    mag_ref[...] = jnp.where(fi_iota < _F, mag, -1.0)
    sr_ref[0] = jnp.zeros((_FP, _DBLK), jnp.float32)
    si_ref[0] = jnp.zeros((_FP, _DBLK), jnp.float32)

    def step(k, _):
        m = mag_ref[...]
        mx = jnp.max(m, axis=0, keepdims=True)
        cand = jnp.where(m == mx, fi_iota, jnp.int32(1 << 30))
        am = jnp.min(cand, axis=0, keepdims=True)
        win = fi_iota == am
        mag_ref[...] = jnp.where(win, -1.0, m)
        a = jnp.where(win, xr, 0.0)
        b = jnp.where(win, xi, 0.0)
        frk = fr_ref[pl.ds(k, 1), :]
        fik = fi_ref[pl.ds(k, 1), :]
        sr_ref[0] = sr_ref[0] + a * frk - b * fik
        si_ref[0] = si_ref[0] + a * fik + b * frk
        return 0

    lax.fori_loop(0, _K, step, 0)


def _select_filter(xr, xi, filter_real, filter_imag):
    B = xr.shape[0]
    return pl.pallas_call(
        _select_body,
        grid=(B, _D // _DBLK),
        in_specs=[
            pl.BlockSpec((1, _FP, _DBLK), lambda b, d: (b, 0, d)),
            pl.BlockSpec((1, _FP, _DBLK), lambda b, d: (b, 0, d)),
            pl.BlockSpec((_K, _DBLK), lambda b, d: (0, d)),
            pl.BlockSpec((_K, _DBLK), lambda b, d: (0, d)),
        ],
        out_specs=[
            pl.BlockSpec((1, _FP, _DBLK), lambda b, d: (b, 0, d)),
            pl.BlockSpec((1, _FP, _DBLK), lambda b, d: (b, 0, d)),
        ],
        out_shape=[
            jax.ShapeDtypeStruct((B, _FP, _D), jnp.float32),
            jax.ShapeDtypeStruct((B, _FP, _D), jnp.float32),
        ],
        scratch_shapes=[pltpu.VMEM((_FP, _DBLK), jnp.float32)],
    )(xr, xi, filter_real, filter_imag)


def _inv_body(br_ref, bi_ref, sr_ref, si_ref, out_ref):
    acc = jnp.dot(br_ref[...], sr_ref[0], preferred_element_type=jnp.float32,
                  precision=_HI)
    acc = acc + jnp.dot(bi_ref[...], si_ref[0],
                        preferred_element_type=jnp.float32, precision=_HI)
    out_ref[0] = acc


def _inv_dft(sr, si, br, bi):
    B = sr.shape[0]
    return pl.pallas_call(
        _inv_body,
        grid=(B, _N // _NBLK, _D // _DBLK),
        in_specs=[
            pl.BlockSpec((_NBLK, _FP), lambda b, n, d: (n, 0)),
            pl.BlockSpec((_NBLK, _FP), lambda b, n, d: (n, 0)),
            pl.BlockSpec((1, _FP, _DBLK), lambda b, n, d: (b, 0, d)),
            pl.BlockSpec((1, _FP, _DBLK), lambda b, n, d: (b, 0, d)),
        ],
        out_specs=pl.BlockSpec((1, _NBLK, _DBLK), lambda b, n, d: (b, n, d)),
        out_shape=jax.ShapeDtypeStruct((B, _N, _D), jnp.float32),
    )(br, bi, sr, si)


def kernel(x, filter_real, filter_imag):
    ct = jnp.asarray(_CT)
    st = jnp.asarray(_ST)
    br = jnp.asarray(_BR)
    bi = jnp.asarray(_BI)

    xr, xi = _fwd_dft(x, ct, st)
    sr, si = _select_filter(xr, xi, filter_real, filter_imag)
    return _inv_dft(sr, si, br, bi)


# speedup vs baseline: 5.3769x; 1.1437x over previous
import jax
import jax.numpy as jnp
import numpy as np
from jax import lax
from jax.experimental import pallas as pl
from jax.experimental.pallas import tpu as pltpu

_N = 8192
_D = 768
_F = _N // 2 + 1
_FP = 4352
_K = 256

_HI = jax.lax.Precision.HIGHEST


def _np_bases():
    n = np.arange(_N, dtype=np.float64)
    f = np.arange(_FP, dtype=np.float64)
    ang = (2.0 * np.pi / _N) * np.outer(f, n)
    ct = np.cos(ang)
    st = -np.sin(ang)
    wr = np.full(_FP, 2.0)
    wr[0] = 1.0
    wr[_F - 1] = 1.0
    wr[_F:] = 0.0
    wi = np.full(_FP, 2.0)
    wi[0] = 0.0
    wi[_F - 1 :] = 0.0
    br = (wr[None, :] / _N) * np.cos(ang.T)
    bi = -(wi[None, :] / _N) * np.sin(ang.T)
    return (ct.astype(np.float32), st.astype(np.float32),
            br.astype(np.float32), bi.astype(np.float32))


_CT, _ST, _BR, _BI = _np_bases()

_DBLK = 128
_FBLK = 256
_NBLK = 512


def _fwd_body(ct_ref, st_ref, x_ref, xr_ref, xi_ref):
    x = x_ref[0]
    xr_ref[0] = jnp.dot(ct_ref[...], x, preferred_element_type=jnp.float32,
                        precision=_HI)
    xi_ref[0] = jnp.dot(st_ref[...], x, preferred_element_type=jnp.float32,
                        precision=_HI)


def _fwd_dft(x, ct, st):
    B = x.shape[0]
    return pl.pallas_call(
        _fwd_body,
        grid=(B, _FP // _FBLK, _D // _DBLK),
        in_specs=[
            pl.BlockSpec((_FBLK, _N), lambda b, f, d: (f, 0)),
            pl.BlockSpec((_FBLK, _N), lambda b, f, d: (f, 0)),
            pl.BlockSpec((1, _N, _DBLK), lambda b, f, d: (b, 0, d)),
        ],
        out_specs=[
            pl.BlockSpec((1, _FBLK, _DBLK), lambda b, f, d: (b, f, d)),
            pl.BlockSpec((1, _FBLK, _DBLK), lambda b, f, d: (b, f, d)),
        ],
        out_shape=[
            jax.ShapeDtypeStruct((B, _FP, _D), jnp.float32),
            jax.ShapeDtypeStruct((B, _FP, _D), jnp.float32),
        ],
    )(ct, st, x)


_FS = 4104


def _select_body(xr_ref, xi_ref, fr_ref, fi_ref, sr_ref, si_ref, mag_ref):
    xr = xr_ref[0, : _FS]
    xi = xi_ref[0, : _FS]
    fi_iota = lax.broadcasted_iota(jnp.int32, (_FS, _DBLK), 0)
    mag = jnp.sqrt(xr * xr + xi * xi)
    mag_ref[...] = jnp.where(fi_iota < _F, mag, -1.0)
    sr_ref[0] = jnp.zeros((_FP, _DBLK), jnp.float32)
    si_ref[0] = jnp.zeros((_FP, _DBLK), jnp.float32)

    def step(k, _):
        m = mag_ref[...]
        am = jnp.argmax(m, axis=0)[None, :]
        win = fi_iota == am
        mag_ref[...] = jnp.where(win, -1.0, m)
        a = jnp.where(win, xr, 0.0)
        b = jnp.where(win, xi, 0.0)
        frk = fr_ref[pl.ds(k, 1), :]
        fik = fi_ref[pl.ds(k, 1), :]
        sr_ref[0, : _FS] = sr_ref[0, : _FS] + a * frk - b * fik
        si_ref[0, : _FS] = si_ref[0, : _FS] + a * fik + b * frk
        return 0

    lax.fori_loop(0, _K, step, 0)


def _select_filter(xr, xi, filter_real, filter_imag):
    B = xr.shape[0]
    return pl.pallas_call(
        _select_body,
        grid=(B, _D // _DBLK),
        in_specs=[
            pl.BlockSpec((1, _FP, _DBLK), lambda b, d: (b, 0, d)),
            pl.BlockSpec((1, _FP, _DBLK), lambda b, d: (b, 0, d)),
            pl.BlockSpec((_K, _DBLK), lambda b, d: (0, d)),
            pl.BlockSpec((_K, _DBLK), lambda b, d: (0, d)),
        ],
        out_specs=[
            pl.BlockSpec((1, _FP, _DBLK), lambda b, d: (b, 0, d)),
            pl.BlockSpec((1, _FP, _DBLK), lambda b, d: (b, 0, d)),
        ],
        out_shape=[
            jax.ShapeDtypeStruct((B, _FP, _D), jnp.float32),
            jax.ShapeDtypeStruct((B, _FP, _D), jnp.float32),
        ],
        scratch_shapes=[pltpu.VMEM((_FS, _DBLK), jnp.float32)],
    )(xr, xi, filter_real, filter_imag)


def _inv_body(br_ref, bi_ref, sr_ref, si_ref, out_ref):
    acc = jnp.dot(br_ref[...], sr_ref[0], preferred_element_type=jnp.float32,
                  precision=_HI)
    acc = acc + jnp.dot(bi_ref[...], si_ref[0],
                        preferred_element_type=jnp.float32, precision=_HI)
    out_ref[0] = acc


def _inv_dft(sr, si, br, bi):
    B = sr.shape[0]
    return pl.pallas_call(
        _inv_body,
        grid=(B, _N // _NBLK, _D // _DBLK),
        in_specs=[
            pl.BlockSpec((_NBLK, _FP), lambda b, n, d: (n, 0)),
            pl.BlockSpec((_NBLK, _FP), lambda b, n, d: (n, 0)),
            pl.BlockSpec((1, _FP, _DBLK), lambda b, n, d: (b, 0, d)),
            pl.BlockSpec((1, _FP, _DBLK), lambda b, n, d: (b, 0, d)),
        ],
        out_specs=pl.BlockSpec((1, _NBLK, _DBLK), lambda b, n, d: (b, n, d)),
        out_shape=jax.ShapeDtypeStruct((B, _N, _D), jnp.float32),
    )(br, bi, sr, si)


def kernel(x, filter_real, filter_imag):
    ct = jnp.asarray(_CT)
    st = jnp.asarray(_ST)
    br = jnp.asarray(_BR)
    bi = jnp.asarray(_BI)

    xr, xi = _fwd_dft(x, ct, st)
    sr, si = _select_filter(xr, xi, filter_real, filter_imag)
    return _inv_dft(sr, si, br, bi)
